# baseline (speedup 1.0000x reference)
import functools

import jax
import jax.numpy as jnp
from jax import lax
from jax.experimental import pallas as pl
from jax.experimental.pallas import tpu as pltpu
from jax.experimental.pallas import tpu_sc as plsc

_B = 256
_P = 512 * 512
_NPX = 256
_NC, _NS = 2, 16
_NW = _NC * _NS
_BPW = _B // _NW
_CHUNK = 128
_ROWS = _BPW * (_NPX // _CHUNK)
_LANES = 16

_mesh = plsc.VectorSubcoreMesh(core_axis_name="c", subcore_axis_name="s")


@functools.partial(
    pl.kernel,
    mesh=_mesh,
    out_type=jax.ShapeDtypeStruct((_B * _NPX,), jnp.float32),
    scratch_types=[
        pltpu.VMEM((_NPX,), jnp.int32),
        pltpu.VMEM((_ROWS, _CHUNK), jnp.int32),
        pltpu.VMEM((_ROWS * _CHUNK,), jnp.float32),
        pltpu.SemaphoreType.DMA,
    ],
)
def _sc_gather(img_hbm, px_hbm, out_hbm, px_v, idx_v, val_v, sem):
    wid = lax.axis_index("s") * _NC + lax.axis_index("c")
    base_b = wid * _BPW
    pltpu.sync_copy(px_hbm, px_v)
    for r in range(_ROWS):
        b = r // (_NPX // _CHUNK)
        half = r % (_NPX // _CHUNK)
        off = (base_b + b) * _P
        for c in range(_CHUNK // _LANES):
            src = pl.ds(half * _CHUNK + c * _LANES, _LANES)
            dst = pl.ds(c * _LANES, _LANES)
            idx_v[r, dst] = px_v[src] + off
    copies = [
        pltpu.async_copy(
            img_hbm.at[idx_v.at[r]],
            val_v.at[pl.ds(r * _CHUNK, _CHUNK)],
            sem,
        )
        for r in range(_ROWS)
    ]
    for cp in copies:
        cp.wait()
    pltpu.sync_copy(val_v, out_hbm.at[pl.ds(wid * _ROWS * _CHUNK, _ROWS * _CHUNK)])


def kernel(img, px_ind):
    batch = img.shape[0]
    img_flat = img.reshape(-1)
    out = _sc_gather(img_flat, px_ind)
    return out.reshape(batch, _NPX)


# speedup vs baseline: 4.4707x; 4.4707x over previous
import functools

import jax
import jax.numpy as jnp
from jax import lax
from jax.experimental import pallas as pl
from jax.experimental.pallas import tpu as pltpu
from jax.experimental.pallas import tpu_sc as plsc

_B = 256
_H = 512
_W = 512
_NPX = 256
_NC, _NS = 2, 16
_NW = _NC * _NS
_BPW = _B // _NW
_WAVE = 128
_NWAVES = _BPW * _NPX // _WAVE
_LANES = 16
_GRAN = 128

_mesh = plsc.VectorSubcoreMesh(core_axis_name="c", subcore_axis_name="s")


@functools.partial(
    pl.kernel,
    mesh=_mesh,
    out_type=jax.ShapeDtypeStruct((_B, _NPX), jnp.float32),
    scratch_types=[
        pltpu.VMEM((_NPX,), jnp.int32),
        pltpu.VMEM((_NPX,), jnp.int32),
        pltpu.VMEM((_WAVE,), jnp.int32),
        pltpu.VMEM((_WAVE, _GRAN), jnp.float32),
        pltpu.VMEM((_WAVE, _W), jnp.float32),
        pltpu.VMEM((_BPW, _NPX), jnp.float32),
        pltpu.SemaphoreType.DMA,
    ],
    compiler_params=pltpu.CompilerParams(needs_layout_passes=False),
)
def _sc_gather(img_hbm, px_hbm, out_hbm, row_pat_v, col_v, row_v, gran_v,
               rows_v, out_v, sem):
    wid = lax.axis_index("s") * _NC + lax.axis_index("c")
    base_b = wid * _BPW
    pltpu.sync_copy(px_hbm, row_pat_v)
    lane_iota = lax.iota(jnp.int32, _LANES)

    cmin = jnp.full((_LANES,), _W - 1, jnp.int32)
    cmax = jnp.zeros((_LANES,), jnp.int32)
    for k in range(_NPX // _LANES):
        sl = pl.ds(k * _LANES, _LANES)
        px = row_pat_v[sl]
        col = px & (_W - 1)
        cmin = jnp.minimum(cmin, col)
        cmax = jnp.maximum(cmax, col)
        col_v[sl] = col
        row_pat_v[sl] = px >> 9
    cmin_s = jnp.min(cmin, axis=0)
    cmax_s = jnp.max(cmax, axis=0)
    win0 = pl.multiple_of((cmin_s >> 7) << 7, _GRAN)
    one_window = (cmax_s >> 7) == (cmin_s >> 7)

    @pl.when(one_window)
    def _fast():
        @pl.loop(0, _NWAVES)
        def _wave(w):
            b = w >> 1
            j0 = (w & 1) * _WAVE
            row_base = (base_b + b) * _H
            for k in range(_WAVE // _LANES):
                sl = pl.ds(k * _LANES, _LANES)
                row_v[sl] = row_pat_v[pl.ds(j0 + k * _LANES, _LANES)] + row_base
            pltpu.async_copy(
                img_hbm.at[row_v, pl.ds(win0, _GRAN)], gran_v, sem
            ).wait()
            for k in range(_WAVE // _LANES):
                lane = col_v[pl.ds(j0 + k * _LANES, _LANES)] - win0
                vals = plsc.load_gather(
                    gran_v, [lane_iota + k * _LANES, lane]
                )
                out_v[b, pl.ds(j0 + k * _LANES, _LANES)] = vals

    @pl.when(jnp.logical_not(one_window))
    def _general():
        @pl.loop(0, _NWAVES)
        def _wave(w):
            b = w >> 1
            j0 = (w & 1) * _WAVE
            row_base = (base_b + b) * _H
            for k in range(_WAVE // _LANES):
                sl = pl.ds(k * _LANES, _LANES)
                row_v[sl] = row_pat_v[pl.ds(j0 + k * _LANES, _LANES)] + row_base
            pltpu.async_copy(img_hbm.at[row_v], rows_v, sem).wait()
            for k in range(_WAVE // _LANES):
                vals = plsc.load_gather(
                    rows_v,
                    [lane_iota + k * _LANES, col_v[pl.ds(j0 + k * _LANES, _LANES)]],
                )
                out_v[b, pl.ds(j0 + k * _LANES, _LANES)] = vals

    pltpu.sync_copy(out_v, out_hbm.at[pl.ds(base_b, _BPW)])


def kernel(img, px_ind):
    img2 = img.reshape(_B * _H, _W)
    return _sc_gather(img2, px_ind)


# speedup vs baseline: 5.5974x; 1.2520x over previous
import functools

import jax
import jax.numpy as jnp
from jax import lax
from jax.experimental import pallas as pl
from jax.experimental.pallas import tpu as pltpu
from jax.experimental.pallas import tpu_sc as plsc

_B = 256
_H = 512
_W = 512
_NPX = 256
_NC, _NS = 2, 16
_NW = _NC * _NS
_BPW = _B // _NW
_WAVE = 128
_NWAVES = _BPW * _NPX // _WAVE
_LANES = 16
_GRAN = 128

_mesh = plsc.VectorSubcoreMesh(core_axis_name="c", subcore_axis_name="s")


@functools.partial(
    pl.kernel,
    mesh=_mesh,
    out_type=jax.ShapeDtypeStruct((_B, _NPX), jnp.float32),
    scratch_types=[
        pltpu.VMEM((_NPX,), jnp.int32),
        pltpu.VMEM((_NPX,), jnp.int32),
        pltpu.VMEM((_WAVE,), jnp.int32),
        pltpu.VMEM((_WAVE,), jnp.int32),
        pltpu.VMEM((_WAVE, _GRAN), jnp.float32),
        pltpu.VMEM((_WAVE, _GRAN), jnp.float32),
        pltpu.VMEM((_WAVE, _W), jnp.float32),
        pltpu.VMEM((_BPW, _NPX), jnp.float32),
        pltpu.SemaphoreType.DMA,
        pltpu.SemaphoreType.DMA,
    ],
    compiler_params=pltpu.CompilerParams(needs_layout_passes=False),
)
def _sc_gather(img_hbm, px_hbm, out_hbm, row_pat_v, col_v, row_v0, row_v1,
               gran_v0, gran_v1, rows_v, out_v, sem0, sem1):
    wid = lax.axis_index("s") * _NC + lax.axis_index("c")
    base_b = wid * _BPW
    pltpu.sync_copy(px_hbm, row_pat_v)
    lane_iota = lax.iota(jnp.int32, _LANES)

    cmin = jnp.full((_LANES,), _W - 1, jnp.int32)
    cmax = jnp.zeros((_LANES,), jnp.int32)
    for k in range(_NPX // _LANES):
        sl = pl.ds(k * _LANES, _LANES)
        px = row_pat_v[sl]
        col = px & (_W - 1)
        cmin = jnp.minimum(cmin, col)
        cmax = jnp.maximum(cmax, col)
        col_v[sl] = col
        row_pat_v[sl] = px >> 9
    cmin_s = jnp.min(cmin, axis=0)
    cmax_s = jnp.max(cmax, axis=0)
    win0 = pl.multiple_of((cmin_s >> 7) << 7, _GRAN)
    one_window = (cmax_s >> 7) == (cmin_s >> 7)

    row_ring = (row_v0, row_v1)
    gran_ring = (gran_v0, gran_v1)
    sem_ring = (sem0, sem1)

    def _build_and_fire(w, slot):
        b = w >> 1
        j0 = (w & 1) * _WAVE
        row_base = (base_b + b) * _H
        rv = row_ring[slot]
        for k in range(_WAVE // _LANES):
            sl = pl.ds(k * _LANES, _LANES)
            rv[sl] = row_pat_v[pl.ds(j0 + k * _LANES, _LANES)] + row_base
        pltpu.async_copy(
            img_hbm.at[rv, pl.ds(win0, _GRAN)], gran_ring[slot], sem_ring[slot]
        )

    def _drain_and_pick(w, slot):
        b = w >> 1
        j0 = (w & 1) * _WAVE
        pltpu.make_async_copy(
            img_hbm.at[row_ring[slot], pl.ds(win0, _GRAN)],
            gran_ring[slot],
            sem_ring[slot],
        ).wait()
        for k in range(_WAVE // _LANES):
            lane = col_v[pl.ds(j0 + k * _LANES, _LANES)] - win0
            vals = plsc.load_gather(gran_ring[slot], [lane_iota + k * _LANES, lane])
            out_v[b, pl.ds(j0 + k * _LANES, _LANES)] = vals

    @pl.when(one_window)
    def _fast():
        _build_and_fire(jnp.int32(0), 0)

        @pl.loop(0, _NWAVES, step=2)
        def _wave(w0):
            for s in range(2):
                w = w0 + s

                @pl.when(w + 1 < _NWAVES)
                def _fire_next():
                    _build_and_fire(w + 1, (s + 1) % 2)

                _drain_and_pick(w, s)

    @pl.when(jnp.logical_not(one_window))
    def _general():
        @pl.loop(0, _NWAVES)
        def _wave(w):
            b = w >> 1
            j0 = (w & 1) * _WAVE
            row_base = (base_b + b) * _H
            for k in range(_WAVE // _LANES):
                sl = pl.ds(k * _LANES, _LANES)
                row_v0[sl] = row_pat_v[pl.ds(j0 + k * _LANES, _LANES)] + row_base
            pltpu.async_copy(img_hbm.at[row_v0], rows_v, sem0).wait()
            for k in range(_WAVE // _LANES):
                vals = plsc.load_gather(
                    rows_v,
                    [lane_iota + k * _LANES, col_v[pl.ds(j0 + k * _LANES, _LANES)]],
                )
                out_v[b, pl.ds(j0 + k * _LANES, _LANES)] = vals

    pltpu.sync_copy(out_v, out_hbm.at[pl.ds(base_b, _BPW)])


def kernel(img, px_ind):
    img2 = img.reshape(_B * _H, _W)
    return _sc_gather(img2, px_ind)
